# initial kernel scaffold (unmeasured)
import jax
import jax.numpy as jnp
from jax import lax
from jax.experimental import pallas as pl
from jax.experimental.pallas import tpu as pltpu

N_DEV = 4


def _relu_dot(a, w):
    y = lax.dot_general(
        a, w, (((1,), (0,)), ((), ())), preferred_element_type=jnp.float32
    )
    return jnp.maximum(y, 0.0)


def kernel(x, w_mat):
    m_per, k = x.shape
    _, n_per = w_mat.shape

    def body(
        x_ref,
        w_ref,
        out_ref,
        comm_ref,
        amax_tx_ref,
        amax_rx_ref,
        send_sems,
        recv_sems,
        amax_send_sems,
        amax_recv_sems,
    ):
        my = lax.axis_index("i")
        left = lax.rem(my + (N_DEV - 1), N_DEV)
        right = lax.rem(my + 1, N_DEV)

        barrier_sem = pltpu.get_barrier_semaphore()
        for nbr in (left, right):
            pl.semaphore_signal(
                barrier_sem, inc=1,
                device_id=(nbr,), device_id_type=pl.DeviceIdType.MESH,
            )
        pl.semaphore_wait(barrier_sem, 2)

        def hop_rdma(h):
            src = x_ref if h == 0 else comm_ref.at[h - 1]
            return pltpu.make_async_remote_copy(
                src_ref=src,
                dst_ref=comm_ref.at[h],
                send_sem=send_sems.at[h],
                recv_sem=recv_sems.at[h],
                device_id=(right,),
                device_id_type=pl.DeviceIdType.MESH,
            )

        rdma = hop_rdma(0)
        rdma.start()
        out_ref[pl.ds(my * m_per, m_per), :] = _relu_dot(x_ref[...], w_ref[...])
        rdma.wait()

        for h in range(1, N_DEV - 1):
            rdma = hop_rdma(h)
            rdma.start()
            origin = lax.rem(my + (N_DEV - h), N_DEV)
            out_ref[pl.ds(origin * m_per, m_per), :] = _relu_dot(
                comm_ref[h - 1], w_ref[...]
            )
            rdma.wait()

        origin = lax.rem(my + 1, N_DEV)
        out_ref[pl.ds(origin * m_per, m_per), :] = _relu_dot(
            comm_ref[N_DEV - 2], w_ref[...]
        )

        local_amax = jnp.max(out_ref[...])
        amax_tx_ref[...] = jnp.full((8, 128), local_amax, jnp.float32)

        descs = []
        for off in range(1, N_DEV):
            tgt = lax.rem(my + off, N_DEV)
            d = pltpu.make_async_remote_copy(
                src_ref=amax_tx_ref,
                dst_ref=amax_rx_ref.at[off - 1],
                send_sem=amax_send_sems.at[off - 1],
                recv_sem=amax_recv_sems.at[off - 1],
                device_id=(tgt,),
                device_id_type=pl.DeviceIdType.MESH,
            )
            d.start()
            descs.append(d)
        for d in descs:
            d.wait_send()
            d.wait_recv()

        gmax = jnp.maximum(local_amax, jnp.max(amax_rx_ref[...]))

        scale = gmax / 448.0
        inv_scale = 448.0 / gmax
        q = (out_ref[...] * inv_scale).astype(jnp.float8_e4m3fn)
        out_ref[...] = q.astype(jnp.float32) * scale

    grid_spec = pltpu.PrefetchScalarGridSpec(
        num_scalar_prefetch=0,
        in_specs=[
            pl.BlockSpec(memory_space=pltpu.VMEM),
            pl.BlockSpec(memory_space=pltpu.VMEM),
        ],
        out_specs=pl.BlockSpec(memory_space=pltpu.VMEM),
        scratch_shapes=[
            pltpu.VMEM((N_DEV - 1, m_per, k), x.dtype),
            pltpu.VMEM((8, 128), jnp.float32),
            pltpu.VMEM((N_DEV - 1, 8, 128), jnp.float32),
            pltpu.SemaphoreType.DMA((N_DEV - 1,)),
            pltpu.SemaphoreType.DMA((N_DEV - 1,)),
            pltpu.SemaphoreType.DMA((N_DEV - 1,)),
            pltpu.SemaphoreType.DMA((N_DEV - 1,)),
        ],
    )

    return pl.pallas_call(
        body,
        grid_spec=grid_spec,
        out_shape=jax.ShapeDtypeStruct((N_DEV * m_per, n_per), jnp.float32),
        compiler_params=pltpu.CompilerParams(collective_id=0),
    )(x, w_mat)


# baseline (device time: 390954 ns/iter reference)
import jax
import jax.numpy as jnp
from jax import lax
from jax.experimental import pallas as pl
from jax.experimental.pallas import tpu as pltpu

N_DEV = 4


def _relu_dot(a, w):
    y = lax.dot_general(
        a, w, (((1,), (0,)), ((), ())), preferred_element_type=jnp.float32
    )
    return jnp.maximum(y, 0.0)


def kernel(x, w_mat):
    m_per, k = x.shape
    _, n_per = w_mat.shape
    x = x.astype(jnp.bfloat16)
    w_mat = w_mat.astype(jnp.bfloat16)

    def body(
        x_ref,
        w_ref,
        out_hbm,
        comm_ref,
        y_buf,
        amax_tx_ref,
        amax_rx_ref,
        local_sem,
        send_sems,
        recv_sems,
        amax_send_sems,
        amax_recv_sems,
    ):
        my = lax.axis_index("i")
        left = lax.rem(my + (N_DEV - 1), N_DEV)
        right = lax.rem(my + 1, N_DEV)

        barrier_sem = pltpu.get_barrier_semaphore()
        for nbr in (left, right):
            pl.semaphore_signal(
                barrier_sem, inc=1,
                device_id=(nbr,), device_id_type=pl.DeviceIdType.MESH,
            )
        pl.semaphore_wait(barrier_sem, 2)

        def hop_rdma(h):
            src = x_ref if h == 0 else comm_ref.at[h - 1]
            return pltpu.make_async_remote_copy(
                src_ref=src,
                dst_ref=comm_ref.at[h],
                send_sem=send_sems.at[h],
                recv_sem=recv_sems.at[h],
                device_id=(right,),
                device_id_type=pl.DeviceIdType.MESH,
            )

        amax = jnp.float32(0.0)

        n_strips = 4
        n_strip = n_per // n_strips

        def chunk_gemm(c, src_ref, amax):
            origin = lax.rem(my + (N_DEV - c) % N_DEV, N_DEV)
            for t in range(n_strips):
                cols = pl.ds(t * n_strip, n_strip)
                y_buf[:, cols] = _relu_dot(src_ref[...], w_ref[:, cols])
                amax = jnp.maximum(amax, jnp.max(y_buf[:, cols]))
            cp = pltpu.make_async_copy(
                y_buf, out_hbm.at[pl.ds(origin * m_per, m_per), :], local_sem
            )
            cp.start()
            cp.wait()
            return amax

        rdma = hop_rdma(0)
        rdma.start()
        amax = chunk_gemm(0, x_ref, amax)
        rdma.wait()
        for h in range(1, N_DEV - 1):
            rdma = hop_rdma(h)
            rdma.start()
            amax = chunk_gemm(h, comm_ref.at[h - 1], amax)
            rdma.wait()
        amax = chunk_gemm(N_DEV - 1, comm_ref.at[N_DEV - 2], amax)

        amax_tx_ref[...] = jnp.full((8, 128), amax, jnp.float32)
        descs = []
        for off in range(1, N_DEV):
            tgt = lax.rem(my + off, N_DEV)
            d = pltpu.make_async_remote_copy(
                src_ref=amax_tx_ref,
                dst_ref=amax_rx_ref.at[off - 1],
                send_sem=amax_send_sems.at[off - 1],
                recv_sem=amax_recv_sems.at[off - 1],
                device_id=(tgt,),
                device_id_type=pl.DeviceIdType.MESH,
            )
            d.start()
            descs.append(d)
        for d in descs:
            d.wait_send()
            d.wait_recv()
        gmax = jnp.maximum(amax, jnp.max(amax_rx_ref[...]))

        scale = gmax / 448.0
        inv_scale = 448.0 / gmax
        for b in range(N_DEV):
            rows = pl.ds(b * m_per, m_per)
            cp = pltpu.make_async_copy(out_hbm.at[rows, :], y_buf, local_sem)
            cp.start()
            cp.wait()
            for t in range(n_strips):
                cols = pl.ds(t * n_strip, n_strip)
                q = (y_buf[:, cols] * inv_scale).astype(jnp.float8_e4m3fn)
                y_buf[:, cols] = q.astype(jnp.float32) * scale
            cp = pltpu.make_async_copy(y_buf, out_hbm.at[rows, :], local_sem)
            cp.start()
            cp.wait()

    grid_spec = pltpu.PrefetchScalarGridSpec(
        num_scalar_prefetch=0,
        in_specs=[
            pl.BlockSpec(memory_space=pltpu.VMEM),
            pl.BlockSpec(memory_space=pltpu.VMEM),
        ],
        out_specs=pl.BlockSpec(memory_space=pl.ANY),
        scratch_shapes=[
            pltpu.VMEM((N_DEV - 1, m_per, k), jnp.bfloat16),
            pltpu.VMEM((m_per, n_per), jnp.float32),
            pltpu.VMEM((8, 128), jnp.float32),
            pltpu.VMEM((N_DEV - 1, 8, 128), jnp.float32),
            pltpu.SemaphoreType.DMA,
            pltpu.SemaphoreType.DMA((N_DEV - 1,)),
            pltpu.SemaphoreType.DMA((N_DEV - 1,)),
            pltpu.SemaphoreType.DMA((N_DEV - 1,)),
            pltpu.SemaphoreType.DMA((N_DEV - 1,)),
        ],
    )

    return pl.pallas_call(
        body,
        grid_spec=grid_spec,
        out_shape=jax.ShapeDtypeStruct((N_DEV * m_per, n_per), jnp.float32),
        compiler_params=pltpu.CompilerParams(
            collective_id=0, vmem_limit_bytes=60 * 1024 * 1024
        ),
    )(x, w_mat)
